# baseline (speedup 1.0000x reference)
import jax
import jax.numpy as jnp
from jax import lax
from jax.experimental import pallas as pl
from jax.experimental.pallas import tpu as pltpu
from jax.experimental.pallas import tpu_sc as plsc

RES = 128
THRESH = 0.01
N = 2097152
N_CELLS = RES * RES * RES
PAD = 8192
TOT = N_CELLS + PAD

NUM_TILES = 16
NPT = N // NUM_TILES
CHUNK = 8192
ROWS = CHUNK // 128
NCHUNK = NPT // CHUNK


def _body(coords_ref, dens_ref, grid_ref, out_ref,
          xv, yv, zv, dv, idxbuf0, idxbuf1, ones, sem_in, sem_scat):
    sid = lax.axis_index("s")

    @pl.loop(0, CHUNK // 16)
    def _fill(r):
        ones[pl.ds(r * 16, 16)] = jnp.full((16,), 1.0, jnp.float32)

    slab = N_CELLS // NUM_TILES
    pltpu.sync_copy(grid_ref.at[pl.ds(sid * slab, slab)],
                    out_ref.at[pl.ds(sid * slab, slab)])
    padslab = PAD // NUM_TILES
    pltpu.sync_copy(grid_ref.at[pl.ds(sid * padslab, padslab)],
                    out_ref.at[pl.ds(N_CELLS + sid * padslab, padslab)])
    plsc.subcore_barrier()

    lane = lax.iota(jnp.int32, 16)

    def start_in(k):
        b = k % 2
        base = sid * NPT + k * CHUNK
        pltpu.async_copy(coords_ref.at[pl.ds(base, CHUNK)],
                         xv.at[b], sem_in.at[b])
        pltpu.async_copy(coords_ref.at[pl.ds(N + base, CHUNK)],
                         yv.at[b], sem_in.at[b])
        pltpu.async_copy(coords_ref.at[pl.ds(2 * N + base, CHUNK)],
                         zv.at[b], sem_in.at[b])
        pltpu.async_copy(dens_ref.at[pl.ds(base, CHUNK)],
                         dv.at[b], sem_in.at[b])

    def wait_in(k):
        b = k % 2
        pltpu.make_async_copy(coords_ref.at[pl.ds(0, CHUNK)],
                              xv.at[b], sem_in.at[b]).wait()
        pltpu.make_async_copy(coords_ref.at[pl.ds(0, CHUNK)],
                              yv.at[b], sem_in.at[b]).wait()
        pltpu.make_async_copy(coords_ref.at[pl.ds(0, CHUNK)],
                              zv.at[b], sem_in.at[b]).wait()
        pltpu.make_async_copy(dens_ref.at[pl.ds(0, CHUNK)],
                              dv.at[b], sem_in.at[b]).wait()

    def compute(k):
        b = k % 2
        idxbuf = idxbuf0 if b == 0 else idxbuf1

        @pl.loop(0, ROWS)
        def _row(r):
            for g in range(8):
                off = r * 128 + g * 16
                pid = off + lane
                x = xv[b, pl.ds(off, 16)]
                y = yv[b, pl.ds(off, 16)]
                z = zv[b, pl.ds(off, 16)]
                ix = jnp.clip((x * 127.0).astype(jnp.int32), 0, RES - 1)
                iy = jnp.clip((y * 127.0).astype(jnp.int32), 0, RES - 1)
                iz = jnp.clip((z * 127.0).astype(jnp.int32), 0, RES - 1)
                lin = (ix * RES + iy) * RES + iz
                d = dv[b, pl.ds(off, 16)]
                trash = N_CELLS + (pid & (PAD - 1))
                idxbuf[pl.ds(off, 16)] = jnp.where(d > THRESH, lin, trash)

    def fire_scat(k):
        b = k % 2
        idxbuf = idxbuf0 if b == 0 else idxbuf1
        pltpu.async_copy(ones, out_ref.at[idxbuf], sem_scat.at[b])

    def drain_scat(k):
        b = k % 2
        idxbuf = idxbuf0 if b == 0 else idxbuf1
        pltpu.make_async_copy(ones, out_ref.at[idxbuf],
                              sem_scat.at[b]).wait()

    start_in(0)
    for k in range(NCHUNK):
        wait_in(k)
        if k + 1 < NCHUNK:
            start_in(k + 1)
        if k >= 2:
            drain_scat(k - 2)
        compute(k)
        fire_scat(k)
    drain_scat(NCHUNK - 2)
    drain_scat(NCHUNK - 1)


_mesh = plsc.VectorSubcoreMesh(
    core_axis_name="c", subcore_axis_name="s", num_cores=1)

_scatter = pl.kernel(
    _body,
    out_type=jax.ShapeDtypeStruct((TOT,), jnp.float32),
    mesh=_mesh,
    scratch_types=[
        pltpu.VMEM((2, CHUNK), jnp.float32),
        pltpu.VMEM((2, CHUNK), jnp.float32),
        pltpu.VMEM((2, CHUNK), jnp.float32),
        pltpu.VMEM((2, CHUNK), jnp.float32),
        pltpu.VMEM((CHUNK,), jnp.int32),
        pltpu.VMEM((CHUNK,), jnp.int32),
        pltpu.VMEM((CHUNK,), jnp.float32),
        pltpu.SemaphoreType.DMA((2,)),
        pltpu.SemaphoreType.DMA((2,)),
    ],
)


@jax.jit
def kernel(coords, densities, grid):
    coords_t = coords.T.reshape(-1)
    out = _scatter(coords_t, densities, grid.reshape(-1))
    return out[:N_CELLS].reshape(RES, RES, RES)


# speedup vs baseline: 3.9027x; 3.9027x over previous
import jax
import jax.numpy as jnp
from jax import lax
from jax.experimental import pallas as pl
from jax.experimental.pallas import tpu as pltpu
from jax.experimental.pallas import tpu_sc as plsc

RES = 128
THRESH = 0.01
N = 2097152
N_CELLS = RES * RES * RES
QCELLS = N_CELLS // 4
SPAD = 8192

NUM_TILES = 16
NPT = N // NUM_TILES
CHUNK = 8192
NCHUNK = NPT // CHUNK
QSLAB = QCELLS // NUM_TILES


def _body(coords_ref, dens_ref, grid_ref, out_ref,
          xv, yv, zv, dv, idxbuf0, idxbuf1, ones, gshared,
          sem_in, sem_scat):
    cid = lax.axis_index("c")
    sid = lax.axis_index("s")

    @pl.loop(0, CHUNK // 16)
    def _fill(r):
        ones[pl.ds(r * 16, 16)] = jnp.full((16,), 1.0, jnp.float32)

    lane = lax.iota(jnp.int32, 16)

    def start_in(c, b):
        base = sid * NPT + c * CHUNK
        pltpu.async_copy(coords_ref.at[pl.ds(base, CHUNK)],
                         xv.at[b], sem_in.at[b])
        pltpu.async_copy(coords_ref.at[pl.ds(N + base, CHUNK)],
                         yv.at[b], sem_in.at[b])
        pltpu.async_copy(coords_ref.at[pl.ds(2 * N + base, CHUNK)],
                         zv.at[b], sem_in.at[b])
        pltpu.async_copy(dens_ref.at[pl.ds(base, CHUNK)],
                         dv.at[b], sem_in.at[b])

    def wait_in(b):
        pltpu.make_async_copy(coords_ref.at[pl.ds(0, CHUNK)],
                              xv.at[b], sem_in.at[b]).wait()
        pltpu.make_async_copy(coords_ref.at[pl.ds(0, CHUNK)],
                              yv.at[b], sem_in.at[b]).wait()
        pltpu.make_async_copy(coords_ref.at[pl.ds(0, CHUNK)],
                              zv.at[b], sem_in.at[b]).wait()
        pltpu.make_async_copy(dens_ref.at[pl.ds(0, CHUNK)],
                              dv.at[b], sem_in.at[b]).wait()

    def compute(b, q_base):
        idxbuf = idxbuf0 if b == 0 else idxbuf1

        @pl.loop(0, CHUNK // 128)
        def _row(r):
            for g in range(8):
                off = r * 128 + g * 16
                pid = off + lane
                x = xv[b, pl.ds(off, 16)]
                y = yv[b, pl.ds(off, 16)]
                z = zv[b, pl.ds(off, 16)]
                ix = jnp.clip((x * 127.0).astype(jnp.int32), 0, RES - 1)
                iy = jnp.clip((y * 127.0).astype(jnp.int32), 0, RES - 1)
                iz = jnp.clip((z * 127.0).astype(jnp.int32), 0, RES - 1)
                lin = (ix * RES + iy) * RES + iz
                local = lin - q_base
                d = dv[b, pl.ds(off, 16)]
                keep = (d > THRESH) & (local >= 0) & (local < QCELLS)
                trash = QCELLS + (pid & (SPAD - 1))
                idxbuf[pl.ds(off, 16)] = jnp.where(keep, local, trash)

    def fire_scat(b):
        idxbuf = idxbuf0 if b == 0 else idxbuf1
        pltpu.async_copy(ones, gshared.at[idxbuf], sem_scat.at[b])

    def drain_scat(b):
        idxbuf = idxbuf0 if b == 0 else idxbuf1
        pltpu.make_async_copy(ones, gshared.at[idxbuf],
                              sem_scat.at[b]).wait()

    NJ = NCHUNK // 2

    for p in range(2):
        q_base = (cid * 2 + p) * QCELLS

        pltpu.sync_copy(grid_ref.at[pl.ds(q_base + sid * QSLAB, QSLAB)],
                        gshared.at[pl.ds(sid * QSLAB, QSLAB)])
        plsc.subcore_barrier()

        start_in(0, 0)
        start_in(1, 1)

        @pl.loop(0, NJ)
        def _j(j):
            for h in range(2):
                c = 2 * j + h
                wait_in(h)

                @pl.when(j > 0)
                def _():
                    drain_scat(h)

                compute(h, q_base)
                fire_scat(h)

                @pl.when(j + 1 < NJ)
                def _():
                    start_in(c + 2, h)

        drain_scat(0)
        drain_scat(1)
        plsc.subcore_barrier()

        pltpu.sync_copy(gshared.at[pl.ds(sid * QSLAB, QSLAB)],
                        out_ref.at[pl.ds(q_base + sid * QSLAB, QSLAB)])
        if p == 0:
            plsc.subcore_barrier()


_mesh = plsc.VectorSubcoreMesh(
    core_axis_name="c", subcore_axis_name="s", num_cores=2)

_scatter = pl.kernel(
    _body,
    out_type=jax.ShapeDtypeStruct((N_CELLS,), jnp.float32),
    mesh=_mesh,
    scratch_types=[
        pltpu.VMEM((2, CHUNK), jnp.float32),
        pltpu.VMEM((2, CHUNK), jnp.float32),
        pltpu.VMEM((2, CHUNK), jnp.float32),
        pltpu.VMEM((2, CHUNK), jnp.float32),
        pltpu.VMEM((CHUNK,), jnp.int32),
        pltpu.VMEM((CHUNK,), jnp.int32),
        pltpu.VMEM((CHUNK,), jnp.float32),
        pltpu.VMEM_SHARED((QCELLS + SPAD,), jnp.float32),
        pltpu.SemaphoreType.DMA((2,)),
        pltpu.SemaphoreType.DMA((2,)),
    ],
)


@jax.jit
def kernel(coords, densities, grid):
    coords_t = coords.T.reshape(-1)
    out = _scatter(coords_t, densities, grid.reshape(-1))
    return out.reshape(RES, RES, RES)
